# baseline (speedup 1.0000x reference)
import functools
import numpy as np
import jax
import jax.numpy as jnp
from jax import lax
from jax.experimental import pallas as pl
from jax.experimental.pallas import tpu as pltpu
from jax.experimental.pallas import tpu_sc as plsc

_VOCAB = 100000
_D = 768
_MAX_SEQ = 2048
_SCALE = float(np.sqrt(np.float32(_D)))

_NC = 2
_NS = 16
_NW = _NC * _NS
_LANES = 16

_CHUNK = 16


def _sinus_pe(max_len, d_model):
    pos = np.arange(max_len, dtype=np.float32)[:, None]
    div = np.exp(np.arange(0, d_model, 2, dtype=np.float32) * (-np.log(10000.0) / d_model))
    pe = np.zeros((max_len, d_model), dtype=np.float32)
    pe[:, 0::2] = np.sin(pos * div)
    pe[:, 1::2] = np.cos(pos * div)
    return pe


_PE = _sinus_pe(_MAX_SEQ, _D)


def _make_emb_kernel(batch, seq_len):
    n_rows = batch * seq_len
    assert seq_len % _NW == 0
    pos_per_w = seq_len // _NW
    assert pos_per_w % _CHUNK == 0
    h_per_b = pos_per_w // _CHUNK
    n_chunks = batch * h_per_b
    _NB = 6

    mesh = plsc.VectorSubcoreMesh(
        core_axis_name="c", subcore_axis_name="s",
        num_cores=_NC, num_subcores=_NS)

    @functools.partial(
        pl.kernel,
        out_type=jax.ShapeDtypeStruct((n_rows, _D), jnp.float32),
        mesh=mesh,
        scratch_types=[
            pltpu.VMEM((batch, pos_per_w), jnp.int32),
            [pltpu.VMEM((_CHUNK, _D), jnp.float32) for _ in range(_NB)],
            pltpu.VMEM((pos_per_w, _D), jnp.float32),
            [pltpu.SemaphoreType.DMA for _ in range(_NB)],
            pltpu.SemaphoreType.DMA,
            [pltpu.SemaphoreType.DMA for _ in range(_NB)],
            pltpu.SemaphoreType.DMA,
        ],
    )
    def emb(x_hbm, pe_hbm, table_hbm, out_hbm,
            idx_v, rows_v, pe_v, gsem, psem, osem, isem):
        wid = lax.axis_index("s") * _NC + lax.axis_index("c")
        p_base = wid * pos_per_w
        idx_loads = [
            pltpu.make_async_copy(
                x_hbm.at[b, pl.ds(p_base, pos_per_w)], idx_v.at[b], isem)
            for b in range(batch)
        ]
        for ld in idx_loads:
            ld.start()
        pe_load = pltpu.make_async_copy(
            pe_hbm.at[pl.ds(p_base, pos_per_w)], pe_v, psem)
        pe_load.start()
        for ld in idx_loads:
            ld.wait()

        def gather_copy(c):
            buf = c % _NB
            b, h = c // h_per_b, c % h_per_b
            return pltpu.make_async_copy(
                table_hbm.at[idx_v.at[b, pl.ds(h * _CHUNK, _CHUNK)]],
                rows_v[buf], gsem[buf])

        def store_copy(c):
            buf = c % _NB
            b, h = c // h_per_b, c % h_per_b
            return pltpu.make_async_copy(
                rows_v[buf],
                out_hbm.at[pl.ds(b * seq_len + p_base + h * _CHUNK, _CHUNK)],
                osem[buf])

        depth = _NB - 1
        for k in range(depth):
            gather_copy(k).start()
        pe_load.wait()
        for c in range(n_chunks):
            buf = c % _NB
            gather_copy(c).wait()

            def row_body(r, _, buf=buf, p0=(c % h_per_b) * _CHUNK):
                for j in range(_D // _LANES):
                    sl = pl.ds(j * _LANES, _LANES)
                    rows_v[buf][r, sl] = rows_v[buf][r, sl] * _SCALE + pe_v[p0 + r, sl]
                return 0

            lax.fori_loop(0, _CHUNK, row_body, 0)
            store_copy(c).start()
            if c + depth < n_chunks:
                if c + depth - _NB >= 0:
                    store_copy(c + depth - _NB).wait()
                gather_copy(c + depth).start()
        for c in range(max(0, n_chunks - _NB), n_chunks):
            store_copy(c).wait()

    return emb


@jax.jit
def kernel(x, table):
    batch, seq_len = x.shape
    x2d = x.astype(jnp.int32)
    pe = jnp.asarray(_PE[:seq_len])
    out = _make_emb_kernel(batch, seq_len)(x2d, pe, table)
    return out.reshape(batch, seq_len, _D)


# speedup vs baseline: 1.3553x; 1.3553x over previous
import functools
import numpy as np
import jax
import jax.numpy as jnp
from jax import lax
from jax.experimental import pallas as pl
from jax.experimental.pallas import tpu as pltpu
from jax.experimental.pallas import tpu_sc as plsc

_VOCAB = 100000
_D = 768
_MAX_SEQ = 2048
_SCALE = float(np.sqrt(np.float32(_D)))

_NC = 2
_NS = 16
_NW = _NC * _NS
_LANES = 16

_CHUNK = 32


def _sinus_pe(max_len, d_model):
    pos = np.arange(max_len, dtype=np.float32)[:, None]
    div = np.exp(np.arange(0, d_model, 2, dtype=np.float32) * (-np.log(10000.0) / d_model))
    pe = np.zeros((max_len, d_model), dtype=np.float32)
    pe[:, 0::2] = np.sin(pos * div)
    pe[:, 1::2] = np.cos(pos * div)
    return pe


_PE = _sinus_pe(_MAX_SEQ, _D)


def _make_emb_kernel(batch, seq_len):
    n_rows = batch * seq_len
    assert n_rows % _NW == 0
    rows_per_w = n_rows // _NW
    assert rows_per_w % _CHUNK == 0
    n_chunks = rows_per_w // _CHUNK
    assert seq_len % rows_per_w == 0
    _NB = 3

    mesh = plsc.VectorSubcoreMesh(
        core_axis_name="c", subcore_axis_name="s",
        num_cores=_NC, num_subcores=_NS)

    @functools.partial(
        pl.kernel,
        out_type=jax.ShapeDtypeStruct((n_rows, _D), jnp.float32),
        mesh=mesh,
        scratch_types=[
            pltpu.VMEM((rows_per_w,), jnp.int32),
            [pltpu.VMEM((_CHUNK, _D), jnp.float32) for _ in range(_NB)],
            [pltpu.VMEM((_CHUNK, _D), jnp.float32) for _ in range(2)],
            [pltpu.SemaphoreType.DMA for _ in range(_NB)],
            [pltpu.SemaphoreType.DMA for _ in range(2)],
            [pltpu.SemaphoreType.DMA for _ in range(_NB)],
        ],
    )
    def emb(x_hbm, pe_hbm, table_hbm, out_hbm,
            idx_v, rows_v, pe_v, gsem, psem, osem):
        wid = lax.axis_index("s") * _NC + lax.axis_index("c")
        base = wid * rows_per_w
        s_base = lax.rem(base, seq_len)
        pltpu.sync_copy(x_hbm.at[pl.ds(base, rows_per_w)], idx_v)

        def gather_copy(c):
            buf = c % _NB
            return pltpu.make_async_copy(
                table_hbm.at[idx_v.at[pl.ds(c * _CHUNK, _CHUNK)]],
                rows_v[buf], gsem[buf])

        def pe_copy(c):
            buf = c % 2
            return pltpu.make_async_copy(
                pe_hbm.at[pl.ds(s_base + c * _CHUNK, _CHUNK)],
                pe_v[buf], psem[buf])

        def store_copy(c):
            buf = c % _NB
            return pltpu.make_async_copy(
                rows_v[buf], out_hbm.at[pl.ds(base + c * _CHUNK, _CHUNK)],
                osem[buf])

        gather_copy(0).start()
        pe_copy(0).start()
        gather_copy(1).start()
        pe_copy(1).start()
        for c in range(n_chunks):
            buf = c % _NB
            gather_copy(c).wait()
            pe_copy(c).wait()

            def row_body(r, _, buf=buf, pb=c % 2):
                for j in range(_D // _LANES):
                    sl = pl.ds(j * _LANES, _LANES)
                    rows_v[buf][r, sl] = rows_v[buf][r, sl] * _SCALE + pe_v[pb][r, sl]
                return 0

            lax.fori_loop(0, _CHUNK, row_body, 0)
            store_copy(c).start()
            if c + 2 < n_chunks:
                pe_copy(c + 2).start()
                if c >= 1:
                    store_copy(c - 1).wait()
                gather_copy(c + 2).start()
        store_copy(n_chunks - 3).wait()
        store_copy(n_chunks - 2).wait()
        store_copy(n_chunks - 1).wait()

    return emb


@jax.jit
def kernel(x, table):
    batch, seq_len = x.shape
    x_flat = x.reshape(-1).astype(jnp.int32)
    pe = jnp.asarray(_PE[:seq_len])
    out = _make_emb_kernel(batch, seq_len)(x_flat, pe, table)
    return out.reshape(batch, seq_len, _D)


# speedup vs baseline: 1.3647x; 1.0069x over previous
import functools
import numpy as np
import jax
import jax.numpy as jnp
from jax import lax
from jax.experimental import pallas as pl
from jax.experimental.pallas import tpu as pltpu
from jax.experimental.pallas import tpu_sc as plsc

_VOCAB = 100000
_D = 768
_MAX_SEQ = 2048
_SCALE = float(np.sqrt(np.float32(_D)))

_NC = 2
_NS = 16
_NW = _NC * _NS
_LANES = 16

_CHUNK = 32


def _sinus_pe(max_len, d_model):
    pos = np.arange(max_len, dtype=np.float32)[:, None]
    div = np.exp(np.arange(0, d_model, 2, dtype=np.float32) * (-np.log(10000.0) / d_model))
    pe = np.zeros((max_len, d_model), dtype=np.float32)
    pe[:, 0::2] = np.sin(pos * div)
    pe[:, 1::2] = np.cos(pos * div)
    return pe


_PE = _sinus_pe(_MAX_SEQ, _D)


def _make_emb_kernel(batch, seq_len):
    n_rows = batch * seq_len
    assert n_rows % _NW == 0
    rows_per_w = n_rows // _NW
    assert rows_per_w % _CHUNK == 0
    n_chunks = rows_per_w // _CHUNK
    assert seq_len % rows_per_w == 0
    _NB = 3

    mesh = plsc.VectorSubcoreMesh(
        core_axis_name="c", subcore_axis_name="s",
        num_cores=_NC, num_subcores=_NS)

    @functools.partial(
        pl.kernel,
        out_type=jax.ShapeDtypeStruct((n_rows, _D), jnp.float32),
        mesh=mesh,
        scratch_types=[
            pltpu.VMEM((rows_per_w,), jnp.int32),
            [pltpu.VMEM((_CHUNK, _D), jnp.float32) for _ in range(_NB)],
            [pltpu.VMEM((_CHUNK, _D), jnp.float32) for _ in range(2)],
            [pltpu.SemaphoreType.DMA for _ in range(_NB)],
            [pltpu.SemaphoreType.DMA for _ in range(_NB)],
            [pltpu.SemaphoreType.DMA for _ in range(2)],
            [pltpu.SemaphoreType.DMA for _ in range(_NB)],
        ],
    )
    def emb(x_hbm, pe_hbm, table_hbm, out_hbm,
            idx_v, rows_v, pe_v, gsem, gsem2, psem, osem):
        wid = lax.axis_index("s") * _NC + lax.axis_index("c")
        base = wid * rows_per_w
        s_base = lax.rem(base, seq_len)
        pltpu.sync_copy(x_hbm.at[pl.ds(base, rows_per_w)], idx_v)

        _H = _CHUNK // 2

        def gather_copies(c):
            buf = c % _NB
            return [
                pltpu.make_async_copy(
                    table_hbm.at[idx_v.at[pl.ds(c * _CHUNK, _H)]],
                    rows_v[buf].at[pl.ds(0, _H)], gsem[buf]),
                pltpu.make_async_copy(
                    table_hbm.at[idx_v.at[pl.ds(c * _CHUNK + _H, _H)]],
                    rows_v[buf].at[pl.ds(_H, _H)], gsem2[buf]),
            ]

        def pe_copy(c):
            buf = c % 2
            return pltpu.make_async_copy(
                pe_hbm.at[pl.ds(s_base + c * _CHUNK, _CHUNK)],
                pe_v[buf], psem[buf])

        def store_copy(c):
            buf = c % _NB
            return pltpu.make_async_copy(
                rows_v[buf], out_hbm.at[pl.ds(base + c * _CHUNK, _CHUNK)],
                osem[buf])

        for g in gather_copies(0):
            g.start()
        pe_copy(0).start()
        for g in gather_copies(1):
            g.start()
        pe_copy(1).start()
        for c in range(n_chunks):
            buf = c % _NB
            for g in gather_copies(c):
                g.wait()
            pe_copy(c).wait()

            def row_body(r, _, buf=buf, pb=c % 2):
                for j in range(_D // _LANES):
                    sl = pl.ds(j * _LANES, _LANES)
                    rows_v[buf][r, sl] = rows_v[buf][r, sl] * _SCALE + pe_v[pb][r, sl]
                return 0

            lax.fori_loop(0, _CHUNK, row_body, 0)
            store_copy(c).start()
            if c + 2 < n_chunks:
                pe_copy(c + 2).start()
                if c >= 1:
                    store_copy(c - 1).wait()
                for g in gather_copies(c + 2):
                    g.start()
        store_copy(n_chunks - 3).wait()
        store_copy(n_chunks - 2).wait()
        store_copy(n_chunks - 1).wait()

    return emb


@jax.jit
def kernel(x, table):
    batch, seq_len = x.shape
    x_flat = x.reshape(-1).astype(jnp.int32)
    pe = jnp.asarray(_PE[:seq_len])
    out = _make_emb_kernel(batch, seq_len)(x_flat, pe, table)
    return out.reshape(batch, seq_len, _D)
